# Initial kernel scaffold; baseline (speedup 1.0000x reference)
import jax
import jax.numpy as jnp
from jax.experimental import pallas as pl


def kernel(query, psi_prime, knn_indices, mask, current_coords, all_coords, lam, mu):
    raise NotImplementedError("write your pallas kernel here")


# speedup vs baseline: 301.1371x; 301.1371x over previous
import functools

import jax
import jax.numpy as jnp
from jax import lax
from jax.experimental import pallas as pl
from jax.experimental.pallas import tpu as pltpu
from jax.experimental.pallas import tpu_sc as plsc

B, NP1, K = 32, 20000, 32
C = 2000
G = 16


def _interference_body(psi_t_hbm, knn_hbm, out_hbm, psi_x, psi_y, idx_v, out_v):
    c = lax.axis_index("c")
    s = lax.axis_index("s")
    b = s * 2 + c

    pltpu.sync_copy(psi_t_hbm.at[pl.ds((b * 2) * NP1, NP1)], psi_x)
    pltpu.sync_copy(psi_t_hbm.at[pl.ds((b * 2 + 1) * NP1, NP1)], psi_y)

    iota = lax.broadcasted_iota(jnp.int32, (G,), 0)

    for ci in range(NP1 // C):
        c0 = ci * C
        pltpu.sync_copy(knn_hbm.at[pl.ds((b * NP1 + c0) * K, C * K)], idx_v)

        def group_body(g, carry, c0=c0):
            nloc = g * G
            pos = (nloc + iota) * K
            acc_x = jnp.zeros((G,), jnp.float32)
            acc_y = jnp.zeros((G,), jnp.float32)
            for k in range(K):
                idxv = plsc.load_gather(idx_v, [pos + k])
                acc_x = acc_x + plsc.load_gather(psi_x, [idxv])
                acc_y = acc_y + plsc.load_gather(psi_y, [idxv])
            px = psi_x[pl.ds(c0 + nloc, G)]
            py = psi_y[pl.ds(c0 + nloc, G)]
            out_v[pl.ds(nloc, G)] = px * acc_x + py * acc_y
            return carry

        lax.fori_loop(0, C // G, group_body, 0)
        pltpu.sync_copy(out_v, out_hbm.at[pl.ds(b * NP1 + c0, C)])


@functools.partial(jax.jit, static_argnames=())
def _interference_sc(psi_t, knn):
    mesh = plsc.VectorSubcoreMesh(core_axis_name="c", subcore_axis_name="s")
    fn = functools.partial(
        pl.kernel,
        out_type=jax.ShapeDtypeStruct((B * NP1,), jnp.float32),
        mesh=mesh,
        scratch_types=[
            pltpu.VMEM((NP1,), jnp.float32),
            pltpu.VMEM((NP1,), jnp.float32),
            pltpu.VMEM((C * K,), jnp.int32),
            pltpu.VMEM((C,), jnp.float32),
        ],
        compiler_params=pltpu.CompilerParams(needs_layout_passes=False),
    )(_interference_body)
    return fn(psi_t.reshape(B * 2 * NP1), knn.reshape(B * NP1 * K)).reshape(B, NP1)


def _epilogue_body(q_ref, cur_ref, lam_ref, mu_ref, psi_ref, all_ref, mask_ref,
                   inter_ref, out_ref):
    i = pl.program_id(0)
    qx = q_ref[i, 0]
    qy = q_ref[i, 1]
    cx = cur_ref[i, 0]
    cy = cur_ref[i, 1]
    lam = lam_ref[0]
    mu = mu_ref[0]

    px = psi_ref[0, 0:1, :]
    py = psi_ref[0, 1:2, :]
    ax = all_ref[0, 0:1, :]
    ay = all_ref[0, 1:2, :]
    mk = mask_ref[0, :, :]
    inter = inter_ref[0, :, :]

    dist = jnp.sqrt((ax - cx) ** 2 + (ay - cy) ** 2)
    scores = qx * px + qy * py + lam * inter - mu * dist
    scores = jnp.where(mk > 0.5, jnp.float32(-1e9), scores)
    m = jnp.max(scores)
    e = jnp.exp(scores - m)
    ssum = jnp.sum(e)
    out_ref[0, :, :] = scores - m - jnp.log(ssum)


def _epilogue_tc(query, cur, lam, mu, psi_t, all_t, maskf, inter):
    grid = (B,)
    return pl.pallas_call(
        _epilogue_body,
        grid=grid,
        in_specs=[
            pl.BlockSpec(memory_space=pltpu.SMEM),
            pl.BlockSpec(memory_space=pltpu.SMEM),
            pl.BlockSpec(memory_space=pltpu.SMEM),
            pl.BlockSpec(memory_space=pltpu.SMEM),
            pl.BlockSpec((1, 2, NP1), lambda i: (i, 0, 0)),
            pl.BlockSpec((1, 2, NP1), lambda i: (i, 0, 0)),
            pl.BlockSpec((1, 1, NP1), lambda i: (i, 0, 0)),
            pl.BlockSpec((1, 1, NP1), lambda i: (i, 0, 0)),
        ],
        out_specs=pl.BlockSpec((1, 1, NP1), lambda i: (i, 0, 0)),
        out_shape=jax.ShapeDtypeStruct((B, 1, NP1), jnp.float32),
    )(query, cur, lam, mu, psi_t, all_t,
      maskf.reshape(B, 1, NP1), inter.reshape(B, 1, NP1))


def kernel(query, psi_prime, knn_indices, mask, current_coords, all_coords, lam, mu):
    psi_t = psi_prime.transpose(0, 2, 1)
    all_t = all_coords.transpose(0, 2, 1)
    maskf = mask.astype(jnp.float32)
    inter = _interference_sc(psi_t, knn_indices)
    out = _epilogue_tc(query, current_coords,
                       lam.reshape(1), mu.reshape(1),
                       psi_t, all_t, maskf, inter)
    return out.reshape(B, NP1)
